# baseline (speedup 1.0000x reference)
import numpy as np
import jax
import jax.numpy as jnp
from jax import lax
from jax.experimental import pallas as pl
from jax.experimental.pallas import tpu as pltpu
from jax.experimental.pallas import tpu_sc as plsc

B = 8
N = 4096
D = 2048
K = 10
T = 128
NT = N // T

L = 16
WPR = 4
C = N // WPR
NV = C // L

_NEG = np.float32(-3.0e38)
_POS = np.float32(3.0e38)

_SORT_ROUNDS = []
for _k in (2, 4, 8, 16):
    _d = _k // 2
    while _d >= 1:
        _SORT_ROUNDS.append((_d, _k))
        _d //= 2
_CLEAN_ROUNDS = [8, 4, 2, 1]


def _tc_body(x_ref, wa_ref, ba_ref, scores_ref):
    scores_ref[...] = jnp.sum(x_ref[...] * wa_ref[...], axis=2) + ba_ref[0, 0]


def _tc_scores(x, wa, ba):
    return pl.pallas_call(
        _tc_body,
        grid=(NT,),
        in_specs=[
            pl.BlockSpec((B, T, D), lambda t: (0, t, 0)),
            pl.BlockSpec((1, 1, D), lambda t: (0, 0, 0)),
            pl.BlockSpec((1, 1), lambda t: (0, 0)),
        ],
        out_specs=pl.BlockSpec((B, T), lambda t: (0, t)),
        out_shape=jax.ShapeDtypeStruct((B, N), jnp.float32),
    )(x, wa, ba)


def _masks():
    lanes = lax.iota(jnp.int32, L)
    bit = {d: (lanes & d) != 0 for d in (1, 2, 4, 8)}
    blk = {k: (lanes & k) != 0 for k in (2, 4, 8, 16)}
    sort_tm = [(d, bit[d], jnp.logical_xor(bit[d], blk[k]))
               for d, k in _SORT_ROUNDS]
    clean_tm = [(d, bit[d], bit[d]) for d in _CLEAN_ROUNDS]
    return sort_tm, clean_tm


def _apply_rounds(v, rounds, buf):
    for d, bit_d, take_max in rounds:
        buf[pl.ds(L, L)] = v
        lm = buf[pl.ds(L - d, L)]
        lp = buf[pl.ds(L + d, L)]
        p = jnp.where(bit_d, lm, lp)
        v = jnp.where(take_max, jnp.maximum(v, p), jnp.minimum(v, p))
    return v


def _init_buf(buf):
    zeros = jnp.full((L,), np.float32(0.0), jnp.float32)
    buf[pl.ds(0, L)] = zeros
    buf[pl.ds(2 * L, L)] = zeros


def _sc_body(scores_hbm, wt_hbm, wb_hbm, bc_hbm, ctop_hbm, cbot_hbm, out_hbm,
             chunk_v, stage_top, stage_bot, ct_v, cb_v,
             wt_v, wb_v, bc_v, outv, buf, buf2, buf3, buf4, buf5, buf6):
    c = lax.axis_index("c")
    s = lax.axis_index("s")
    r = c * (B // 2) + s // WPR
    q = s % WPR

    pltpu.sync_copy(scores_hbm.at[r, pl.ds(q * C, C)], chunk_v)

    sort_tm, clean_tm = _masks()
    for b in (buf, buf2, buf3, buf4, buf5, buf6):
        _init_buf(b)

    top0 = jnp.full((L,), _NEG, jnp.float32)
    bot0 = jnp.full((L,), _POS, jnp.float32)

    def body(i, carry):
        top, bot = carry
        v1 = _apply_rounds(chunk_v[pl.ds((2 * i) * L, L)], sort_tm, buf)
        v2 = _apply_rounds(chunk_v[pl.ds((2 * i + 1) * L, L)], sort_tm, buf2)
        rv2 = lax.rev(v2, (0,))
        c_top = _apply_rounds(jnp.maximum(v1, rv2), clean_tm, buf5)
        c_bot = _apply_rounds(jnp.minimum(v1, rv2), clean_tm, buf6)
        top = _apply_rounds(jnp.maximum(top, lax.rev(c_top, (0,))),
                            clean_tm, buf3)
        bot = _apply_rounds(jnp.minimum(bot, lax.rev(c_bot, (0,))),
                            clean_tm, buf4)
        return (top, bot)

    top, bot = lax.fori_loop(0, NV // 2, body, (top0, bot0))

    stage_top[...] = top
    stage_bot[...] = bot
    pltpu.sync_copy(stage_top, ctop_hbm.at[r, q])
    pltpu.sync_copy(stage_bot, cbot_hbm.at[r, q])
    plsc.subcore_barrier()
    plsc.subcore_barrier()

    @pl.when(q == 0)
    def _leader():
        pltpu.sync_copy(ctop_hbm.at[r], ct_v)
        pltpu.sync_copy(cbot_hbm.at[r], cb_v)
        pltpu.sync_copy(wt_hbm, wt_v)
        pltpu.sync_copy(wb_hbm, wb_v)
        pltpu.sync_copy(bc_hbm, bc_v)

        m_top = ct_v[0]
        m_bot = cb_v[0]
        for j in range(1, WPR):
            rv = lax.rev(ct_v[j], (0,))
            m_top = _apply_rounds(jnp.maximum(m_top, rv), clean_tm, buf)
            rv = lax.rev(cb_v[j], (0,))
            m_bot = _apply_rounds(jnp.minimum(m_bot, rv), clean_tm, buf)

        outv[...] = m_top * wt_v[...] + m_bot * wb_v[...] + bc_v[...]
        pltpu.sync_copy(outv, out_hbm.at[r])


def _sc_select(scores, wt, wb, bc):
    mesh = plsc.VectorSubcoreMesh(core_axis_name="c", subcore_axis_name="s")
    kfn = pl.kernel(
        _sc_body,
        mesh=mesh,
        out_type=[
            jax.ShapeDtypeStruct((B, WPR, L), jnp.float32),
            jax.ShapeDtypeStruct((B, WPR, L), jnp.float32),
            jax.ShapeDtypeStruct((B, L), jnp.float32),
        ],
        scratch_types=[
            pltpu.VMEM((C,), jnp.float32),
            pltpu.VMEM((L,), jnp.float32),
            pltpu.VMEM((L,), jnp.float32),
            pltpu.VMEM((WPR, L), jnp.float32),
            pltpu.VMEM((WPR, L), jnp.float32),
            pltpu.VMEM((L,), jnp.float32),
            pltpu.VMEM((L,), jnp.float32),
            pltpu.VMEM((L,), jnp.float32),
            pltpu.VMEM((L,), jnp.float32),
            pltpu.VMEM((3 * L,), jnp.float32),
            pltpu.VMEM((3 * L,), jnp.float32),
            pltpu.VMEM((3 * L,), jnp.float32),
            pltpu.VMEM((3 * L,), jnp.float32),
            pltpu.VMEM((3 * L,), jnp.float32),
            pltpu.VMEM((3 * L,), jnp.float32),
        ],
    )
    _, _, out16 = kfn(scores, wt, wb, bc)
    return out16


def kernel(x, W_attn, b_attn, W_cls, b_cls):
    wa = W_attn.reshape(1, 1, D)
    ba = b_attn.reshape(1, 1)

    idx = np.arange(L)
    wt = jnp.where(idx >= L - K, W_cls[np.clip(15 - idx, 0, 2 * K - 1)], 0.0)
    wb = jnp.where(idx < K, W_cls[np.clip(K + idx, 0, 2 * K - 1)], 0.0)
    bc = jnp.pad(b_cls, (0, L - 1)).astype(jnp.float32)

    scores = _tc_scores(x, wa, ba)
    out16 = _sc_select(scores, wt.astype(jnp.float32),
                       wb.astype(jnp.float32), bc)
    pred = jnp.sum(out16, axis=1, keepdims=True)
    return (pred, scores)


# speedup vs baseline: 1.0082x; 1.0082x over previous
import numpy as np
import jax
import jax.numpy as jnp
from jax import lax
from jax.experimental import pallas as pl
from jax.experimental.pallas import tpu as pltpu
from jax.experimental.pallas import tpu_sc as plsc

B = 8
N = 4096
D = 2048
K = 10
T = 128
NT = N // T

L = 16
WPR = 4
C = N // WPR
NV = C // L

_NEG = np.float32(-3.0e38)
_POS = np.float32(3.0e38)

_SORT_ROUNDS = []
for _k in (2, 4, 8, 16):
    _d = _k // 2
    while _d >= 1:
        _SORT_ROUNDS.append((_d, _k))
        _d //= 2
_CLEAN_ROUNDS = [8, 4, 2, 1]


def _tc_body(x_ref, wa_ref, ba_ref, scores_ref):
    scores_ref[...] = jnp.sum(x_ref[...] * wa_ref[...], axis=2) + ba_ref[0, 0]


def _tc_scores(x, wa, ba):
    return pl.pallas_call(
        _tc_body,
        grid=(NT,),
        in_specs=[
            pl.BlockSpec((B, T, D), lambda t: (0, t, 0)),
            pl.BlockSpec((1, 1, D), lambda t: (0, 0, 0)),
            pl.BlockSpec((1, 1), lambda t: (0, 0)),
        ],
        out_specs=pl.BlockSpec((B, T), lambda t: (0, t)),
        out_shape=jax.ShapeDtypeStruct((B, N), jnp.float32),
    )(x, wa, ba)


def _masks():
    lanes = lax.iota(jnp.int32, L)
    bit = {d: (lanes & d) != 0 for d in (1, 2, 4, 8)}
    blk = {k: (lanes & k) != 0 for k in (2, 4, 8, 16)}
    sort_tm = [(d, bit[d], jnp.logical_xor(bit[d], blk[k]))
               for d, k in _SORT_ROUNDS]
    clean_tm = [(d, bit[d], bit[d]) for d in _CLEAN_ROUNDS]
    return sort_tm, clean_tm


def _apply_rounds(v, rounds, buf):
    for d, bit_d, take_max in rounds:
        buf[pl.ds(L, L)] = v
        lm = buf[pl.ds(L - d, L)]
        lp = buf[pl.ds(L + d, L)]
        p = jnp.where(bit_d, lm, lp)
        v = jnp.where(take_max, jnp.maximum(v, p), jnp.minimum(v, p))
    return v


def _init_buf(buf):
    zeros = jnp.full((L,), np.float32(0.0), jnp.float32)
    buf[pl.ds(0, L)] = zeros
    buf[pl.ds(2 * L, L)] = zeros


def _sc_body(scores_hbm, wt_hbm, wb_hbm, bc_hbm, ctop_hbm, cbot_hbm, out_hbm,
             chunk_v, stage_top, stage_bot, ct_v, cb_v,
             wt_v, wb_v, bc_v, outv, buf, buf2, buf3, buf4, buf5, buf6):
    c = lax.axis_index("c")
    s = lax.axis_index("s")
    r = c * (B // 2) + s // WPR
    q = s % WPR

    pltpu.sync_copy(scores_hbm.at[r, pl.ds(q * C, C)], chunk_v)

    sort_tm, clean_tm = _masks()
    for b in (buf, buf2, buf3, buf4, buf5, buf6):
        _init_buf(b)

    top0 = jnp.full((L,), _NEG, jnp.float32)
    bot0 = jnp.full((L,), _POS, jnp.float32)

    def body(i, carry):
        top, bot = carry
        v1 = _apply_rounds(chunk_v[pl.ds((2 * i) * L, L)], sort_tm, buf)
        v2 = _apply_rounds(chunk_v[pl.ds((2 * i + 1) * L, L)], sort_tm, buf2)
        rv1 = lax.rev(v1, (0,))
        rv2 = lax.rev(v2, (0,))
        top = _apply_rounds(jnp.maximum(top, rv1), clean_tm, buf3)
        bot = _apply_rounds(jnp.minimum(bot, rv1), clean_tm, buf4)
        top = _apply_rounds(jnp.maximum(top, rv2), clean_tm, buf5)
        bot = _apply_rounds(jnp.minimum(bot, rv2), clean_tm, buf6)
        return (top, bot)

    top, bot = lax.fori_loop(0, NV // 2, body, (top0, bot0))

    stage_top[...] = top
    stage_bot[...] = bot
    pltpu.sync_copy(stage_top, ctop_hbm.at[r, q])
    pltpu.sync_copy(stage_bot, cbot_hbm.at[r, q])
    plsc.subcore_barrier()
    plsc.subcore_barrier()

    @pl.when(q == 0)
    def _leader():
        pltpu.sync_copy(ctop_hbm.at[r], ct_v)
        pltpu.sync_copy(cbot_hbm.at[r], cb_v)
        pltpu.sync_copy(wt_hbm, wt_v)
        pltpu.sync_copy(wb_hbm, wb_v)
        pltpu.sync_copy(bc_hbm, bc_v)

        m_top = ct_v[0]
        m_bot = cb_v[0]
        for j in range(1, WPR):
            rv = lax.rev(ct_v[j], (0,))
            m_top = _apply_rounds(jnp.maximum(m_top, rv), clean_tm, buf)
            rv = lax.rev(cb_v[j], (0,))
            m_bot = _apply_rounds(jnp.minimum(m_bot, rv), clean_tm, buf)

        outv[...] = m_top * wt_v[...] + m_bot * wb_v[...] + bc_v[...]
        pltpu.sync_copy(outv, out_hbm.at[r])


def _sc_select(scores, wt, wb, bc):
    mesh = plsc.VectorSubcoreMesh(core_axis_name="c", subcore_axis_name="s")
    kfn = pl.kernel(
        _sc_body,
        mesh=mesh,
        out_type=[
            jax.ShapeDtypeStruct((B, WPR, L), jnp.float32),
            jax.ShapeDtypeStruct((B, WPR, L), jnp.float32),
            jax.ShapeDtypeStruct((B, L), jnp.float32),
        ],
        scratch_types=[
            pltpu.VMEM((C,), jnp.float32),
            pltpu.VMEM((L,), jnp.float32),
            pltpu.VMEM((L,), jnp.float32),
            pltpu.VMEM((WPR, L), jnp.float32),
            pltpu.VMEM((WPR, L), jnp.float32),
            pltpu.VMEM((L,), jnp.float32),
            pltpu.VMEM((L,), jnp.float32),
            pltpu.VMEM((L,), jnp.float32),
            pltpu.VMEM((L,), jnp.float32),
            pltpu.VMEM((3 * L,), jnp.float32),
            pltpu.VMEM((3 * L,), jnp.float32),
            pltpu.VMEM((3 * L,), jnp.float32),
            pltpu.VMEM((3 * L,), jnp.float32),
            pltpu.VMEM((3 * L,), jnp.float32),
            pltpu.VMEM((3 * L,), jnp.float32),
        ],
    )
    _, _, out16 = kfn(scores, wt, wb, bc)
    return out16


def kernel(x, W_attn, b_attn, W_cls, b_cls):
    wa = W_attn.reshape(1, 1, D)
    ba = b_attn.reshape(1, 1)

    idx = np.arange(L)
    wt = jnp.where(idx >= L - K, W_cls[np.clip(15 - idx, 0, 2 * K - 1)], 0.0)
    wb = jnp.where(idx < K, W_cls[np.clip(K + idx, 0, 2 * K - 1)], 0.0)
    bc = jnp.pad(b_cls, (0, L - 1)).astype(jnp.float32)

    scores = _tc_scores(x, wa, ba)
    out16 = _sc_select(scores, wt.astype(jnp.float32),
                       wb.astype(jnp.float32), bc)
    pred = jnp.sum(out16, axis=1, keepdims=True)
    return (pred, scores)
